# baseline (speedup 1.0000x reference)
import functools

import jax
import jax.numpy as jnp
from jax import lax
from jax.experimental import pallas as pl
from jax.experimental.pallas import tpu as pltpu
from jax.experimental.pallas import tpu_sc as plsc

N_NODES = 10000
N_EDGES = 320000
F_IN = 128
HIDDEN = 16
N_CLASSES = 40

NC = 2
NS = 16
LANES = 16
NW = NC * NS

CHUNK = 128
E_ROWS = N_EDGES // CHUNK
N_PAD = 10112
ROWS_PER_SUB = N_PAD // NS


def _sc_mesh():
    return plsc.VectorSubcoreMesh(core_axis_name="c", subcore_axis_name="s")


PROP_N0, PROP_L0 = 112, 72
PROP_N1, PROP_L1 = 48, 28
PROP_BASE1 = 15 * PROP_N0 + PROP_L0
DEG_N, DEG_L0, DEG_L1 = 80, 56, 44
DEG_BASE1 = 15 * DEG_N + DEG_L0
ROWS_MAX = PROP_N0


def _edge_windows(cid, sid, n0, l0, n1, l1, base1):
    last = sid == NS - 1
    nrows = jnp.where(cid == 0,
                      jnp.where(last, l0, n0),
                      jnp.where(last, l1, n1))
    base = jnp.where(cid == 0, sid * n0, base1 + sid * n1)
    return pl.multiple_of(base, 8), nrows


@functools.partial(
    pl.kernel,
    mesh=_sc_mesh(),
    compiler_params=pltpu.CompilerParams(use_tc_tiling_on_sc=False),
    out_type=jax.ShapeDtypeStruct((NC, N_PAD, LANES), jnp.float32),
    scratch_types=[
        pltpu.VMEM((DEG_N, CHUNK), jnp.int32),
        pltpu.VMEM((CHUNK, LANES), jnp.float32),
        pltpu.VMEM_SHARED((N_PAD, LANES), jnp.float32),
    ],
)
def _deg_kernel(e_hbm, ones_hbm, zeros_hbm, out_hbm, dst_v, ones_v, acc):
    cid = lax.axis_index("c")
    sid = lax.axis_index("s")
    r0 = pl.multiple_of(sid * ROWS_PER_SUB, 8)
    base, nrows = _edge_windows(cid, sid, DEG_N, DEG_L0, DEG_N, DEG_L1,
                                DEG_BASE1)
    pltpu.sync_copy(zeros_hbm.at[pl.ds(r0, ROWS_PER_SUB)],
                    acc.at[pl.ds(r0, ROWS_PER_SUB)])
    pltpu.sync_copy(ones_hbm, ones_v)
    is_tail = jnp.logical_and(cid == 1, sid == NS - 1)

    @pl.when(jnp.logical_not(is_tail))
    def _():
        pltpu.sync_copy(e_hbm.at[1, pl.ds(base, DEG_N)], dst_v)

    @pl.when(is_tail)
    def _():
        pltpu.sync_copy(e_hbm.at[1, pl.ds(base, DEG_L1)],
                        dst_v.at[pl.ds(0, DEG_L1)])

    plsc.subcore_barrier()

    def body(j, carry):
        pltpu.sync_copy(ones_v, acc.at[dst_v.at[j]], add=True)
        return carry

    lax.fori_loop(0, nrows, body, 0)
    plsc.subcore_barrier()
    pltpu.sync_copy(acc.at[pl.ds(r0, ROWS_PER_SUB)],
                    out_hbm.at[cid, pl.ds(r0, ROWS_PER_SUB)])


NBUF = 4


def _make_prop_kernel(feat):
    @functools.partial(
        pl.kernel,
        mesh=_sc_mesh(),
        compiler_params=pltpu.CompilerParams(use_tc_tiling_on_sc=False),
        out_type=jax.ShapeDtypeStruct((NC, N_PAD, feat), jnp.float32),
        scratch_types=[
            pltpu.VMEM((ROWS_MAX, CHUNK), jnp.int32),
            pltpu.VMEM((ROWS_MAX, CHUNK), jnp.int32),
            [pltpu.VMEM((CHUNK, feat), jnp.float32) for _ in range(NBUF)],
            pltpu.VMEM_SHARED((N_PAD, feat), jnp.float32),
            [pltpu.SemaphoreType.DMA for _ in range(NBUF)],
        ],
    )
    def prop(h_hbm, e_hbm, zeros_hbm, out_hbm,
             src_v, dst_v, rows, acc, gsems):
        cid = lax.axis_index("c")
        sid = lax.axis_index("s")
        r0 = pl.multiple_of(sid * ROWS_PER_SUB, 8)
        base, nrows = _edge_windows(cid, sid, PROP_N0, PROP_L0, PROP_N1,
                                    PROP_L1, PROP_BASE1)
        pltpu.sync_copy(zeros_hbm.at[pl.ds(r0, ROWS_PER_SUB)],
                        acc.at[pl.ds(r0, ROWS_PER_SUB)])
        on_c0 = cid == 0
        on_c1_body = jnp.logical_and(cid == 1, sid < NS - 1)
        on_c1_tail = jnp.logical_and(cid == 1, sid == NS - 1)

        @pl.when(on_c0)
        def _():
            pltpu.sync_copy(e_hbm.at[0, pl.ds(base, PROP_N0)], src_v)
            pltpu.sync_copy(e_hbm.at[1, pl.ds(base, PROP_N0)], dst_v)

        @pl.when(on_c1_body)
        def _():
            pltpu.sync_copy(e_hbm.at[0, pl.ds(base, PROP_N1)],
                            src_v.at[pl.ds(0, PROP_N1)])
            pltpu.sync_copy(e_hbm.at[1, pl.ds(base, PROP_N1)],
                            dst_v.at[pl.ds(0, PROP_N1)])

        @pl.when(on_c1_tail)
        def _():
            pltpu.sync_copy(e_hbm.at[0, pl.ds(base, PROP_L1)],
                            src_v.at[pl.ds(0, PROP_L1)])
            pltpu.sync_copy(e_hbm.at[1, pl.ds(base, PROP_L1)],
                            dst_v.at[pl.ds(0, PROP_L1)])

        plsc.subcore_barrier()

        for b in range(NBUF):
            pltpu.async_copy(h_hbm.at[src_v.at[b]], rows[b], gsems[b])

        def body(j, carry):
            for b in range(NBUF):
                pltpu.make_async_copy(h_hbm.at[src_v.at[j + b]],
                                      rows[b], gsems[b]).wait()
                pltpu.sync_copy(rows[b], acc.at[dst_v.at[j + b]], add=True)
                pltpu.async_copy(h_hbm.at[src_v.at[j + b + NBUF]],
                                 rows[b], gsems[b])
            return carry

        lax.fori_loop(0, nrows // NBUF - 1, lambda i, c: body(i * NBUF, c), 0)
        j0 = nrows - NBUF
        for b in range(NBUF):
            pltpu.make_async_copy(h_hbm.at[src_v.at[j0 + b]],
                                  rows[b], gsems[b]).wait()
            pltpu.sync_copy(rows[b], acc.at[dst_v.at[j0 + b]], add=True)

        plsc.subcore_barrier()
        pltpu.sync_copy(acc.at[pl.ds(r0, ROWS_PER_SUB)],
                        out_hbm.at[cid, pl.ds(r0, ROWS_PER_SUB)])

    return prop


_prop16 = _make_prop_kernel(HIDDEN)


def _dinv_from_deg(deg_ref):
    deg = deg_ref[0, :, 0:1] + deg_ref[1, :, 0:1]
    return lax.rsqrt(deg + 1.0)


def _tc1_body(x_ref, w_ref, deg_ref, out_ref):
    dinv = _dinv_from_deg(deg_ref)
    h = jnp.dot(x_ref[...], w_ref[...], preferred_element_type=jnp.float32)
    out_ref[0:N_NODES] = h * dinv[0:N_NODES]


_tc1 = pl.pallas_call(
    _tc1_body, out_shape=jax.ShapeDtypeStruct((N_PAD, HIDDEN), jnp.float32))


def _tc2_body(s_ref, h_ref, deg_ref, b_ref, out_ref):
    dinv = lax.rsqrt(deg_ref[0] + deg_ref[1] + 1.0)
    z = dinv * (s_ref[0] + s_ref[1] + h_ref[...]) + b_ref[...]
    out_ref[...] = jnp.maximum(z, 0.0) * dinv


_tc2 = pl.pallas_call(
    _tc2_body, out_shape=jax.ShapeDtypeStruct((N_PAD // 8, 128), jnp.float32))


def _tc3_body(s_ref, h_ref, deg_ref, w_ref, b_ref, out_ref):
    dinv = _dinv_from_deg(deg_ref)
    pz = dinv * (s_ref[0] + s_ref[1] + h_ref[...])
    logits = jnp.dot(pz, w_ref[...], preferred_element_type=jnp.float32) + b_ref[...]
    lane = lax.broadcasted_iota(jnp.int32, logits.shape, 1)
    masked = jnp.where(lane < N_CLASSES, logits, -jnp.inf)
    m = jnp.max(masked, axis=1, keepdims=True)
    lse = jnp.log(jnp.sum(jnp.exp(masked - m), axis=1, keepdims=True)) + m
    out_ref[...] = logits - lse


_tc3 = pl.pallas_call(
    _tc3_body, out_shape=jax.ShapeDtypeStruct((N_PAD, 128), jnp.float32))


def kernel(x, edge_index, W1, b1, W2, b2):
    e3 = edge_index.reshape(2, E_ROWS, CHUNK)
    zeros16 = jnp.zeros((N_PAD, HIDDEN), jnp.float32)
    ones = jnp.ones((CHUNK, LANES), jnp.float32)

    degp = _deg_kernel(e3, ones, zeros16)
    h1s = _tc1(x, W1, degp)
    s1p = _prop16(h1s, e3, zeros16)
    z2s = _tc2(s1p.reshape(NC, N_PAD // 8, 128),
               h1s.reshape(N_PAD // 8, 128),
               degp.reshape(NC, N_PAD // 8, 128),
               jnp.tile(b1, 8).reshape(1, 128)).reshape(N_PAD, HIDDEN)
    s2p = _prop16(z2s, e3, zeros16)
    w2p = jnp.pad(W2, ((0, 0), (0, 128 - N_CLASSES)))
    b2p = jnp.pad(b2, (0, 128 - N_CLASSES)).reshape(1, 128)
    out = _tc3(s2p, z2s, degp, w2p, b2p)
    return out[:N_NODES, :N_CLASSES]


# speedup vs baseline: 1.0032x; 1.0032x over previous
import functools

import jax
import jax.numpy as jnp
from jax import lax
from jax.experimental import pallas as pl
from jax.experimental.pallas import tpu as pltpu
from jax.experimental.pallas import tpu_sc as plsc

N_NODES = 10000
N_EDGES = 320000
F_IN = 128
HIDDEN = 16
N_CLASSES = 40

NC = 2
NS = 16
LANES = 16
NW = NC * NS

CHUNK = 128
E_ROWS = N_EDGES // CHUNK
N_PAD = 10112
ROWS_PER_SUB = N_PAD // NS


def _sc_mesh():
    return plsc.VectorSubcoreMesh(core_axis_name="c", subcore_axis_name="s")


PROP_N0, PROP_L0 = 112, 72
PROP_N1, PROP_L1 = 48, 28
PROP_BASE1 = 15 * PROP_N0 + PROP_L0
DEG_N, DEG_L0, DEG_L1 = 80, 56, 44
DEG_BASE1 = 15 * DEG_N + DEG_L0
ROWS_MAX = PROP_N0


def _edge_windows(cid, sid, n0, l0, n1, l1, base1):
    last = sid == NS - 1
    nrows = jnp.where(cid == 0,
                      jnp.where(last, l0, n0),
                      jnp.where(last, l1, n1))
    base = jnp.where(cid == 0, sid * n0, base1 + sid * n1)
    return pl.multiple_of(base, 8), nrows


@functools.partial(
    pl.kernel,
    mesh=_sc_mesh(),
    compiler_params=pltpu.CompilerParams(use_tc_tiling_on_sc=False),
    out_type=jax.ShapeDtypeStruct((NC, N_PAD, LANES), jnp.float32),
    scratch_types=[
        pltpu.VMEM((DEG_N, CHUNK), jnp.int32),
        pltpu.VMEM((CHUNK, LANES), jnp.float32),
        pltpu.VMEM_SHARED((N_PAD, LANES), jnp.float32),
    ],
)
def _deg_kernel(e_hbm, ones_hbm, zeros_hbm, out_hbm, dst_v, ones_v, acc):
    cid = lax.axis_index("c")
    sid = lax.axis_index("s")
    r0 = pl.multiple_of(sid * ROWS_PER_SUB, 8)
    base, nrows = _edge_windows(cid, sid, DEG_N, DEG_L0, DEG_N, DEG_L1,
                                DEG_BASE1)
    pltpu.sync_copy(zeros_hbm.at[pl.ds(r0, ROWS_PER_SUB)],
                    acc.at[pl.ds(r0, ROWS_PER_SUB)])
    pltpu.sync_copy(ones_hbm, ones_v)
    is_tail = jnp.logical_and(cid == 1, sid == NS - 1)

    @pl.when(jnp.logical_not(is_tail))
    def _():
        pltpu.sync_copy(e_hbm.at[1, pl.ds(base, DEG_N)], dst_v)

    @pl.when(is_tail)
    def _():
        pltpu.sync_copy(e_hbm.at[1, pl.ds(base, DEG_L1)],
                        dst_v.at[pl.ds(0, DEG_L1)])

    plsc.subcore_barrier()

    def body(j, carry):
        pltpu.sync_copy(ones_v, acc.at[dst_v.at[j]], add=True)
        return carry

    lax.fori_loop(0, nrows, body, 0)
    plsc.subcore_barrier()
    pltpu.sync_copy(acc.at[pl.ds(r0, ROWS_PER_SUB)],
                    out_hbm.at[cid, pl.ds(r0, ROWS_PER_SUB)])


NBUF = 4


def _make_prop_kernel(feat):
    @functools.partial(
        pl.kernel,
        mesh=_sc_mesh(),
        compiler_params=pltpu.CompilerParams(use_tc_tiling_on_sc=False),
        out_type=jax.ShapeDtypeStruct((NC, N_PAD, feat), jnp.float32),
        scratch_types=[
            pltpu.VMEM((ROWS_MAX, CHUNK), jnp.int32),
            pltpu.VMEM((ROWS_MAX, CHUNK), jnp.int32),
            [pltpu.VMEM((CHUNK, feat), jnp.float32) for _ in range(NBUF)],
            pltpu.VMEM_SHARED((N_PAD, feat), jnp.float32),
            [pltpu.SemaphoreType.DMA for _ in range(NBUF)],
        ],
    )
    def prop(h_hbm, e_hbm, zeros_hbm, out_hbm,
             src_v, dst_v, rows, acc, gsems):
        cid = lax.axis_index("c")
        sid = lax.axis_index("s")
        r0 = pl.multiple_of(sid * ROWS_PER_SUB, 8)
        base, nrows = _edge_windows(cid, sid, PROP_N0, PROP_L0, PROP_N1,
                                    PROP_L1, PROP_BASE1)
        pltpu.sync_copy(zeros_hbm.at[pl.ds(r0, ROWS_PER_SUB)],
                        acc.at[pl.ds(r0, ROWS_PER_SUB)])
        on_c0 = cid == 0
        on_c1_body = jnp.logical_and(cid == 1, sid < NS - 1)
        on_c1_tail = jnp.logical_and(cid == 1, sid == NS - 1)

        @pl.when(on_c0)
        def _():
            pltpu.sync_copy(e_hbm.at[0, pl.ds(base, PROP_N0)], src_v)
            pltpu.sync_copy(e_hbm.at[1, pl.ds(base, PROP_N0)], dst_v)

        @pl.when(on_c1_body)
        def _():
            pltpu.sync_copy(e_hbm.at[0, pl.ds(base, PROP_N1)],
                            src_v.at[pl.ds(0, PROP_N1)])
            pltpu.sync_copy(e_hbm.at[1, pl.ds(base, PROP_N1)],
                            dst_v.at[pl.ds(0, PROP_N1)])

        @pl.when(on_c1_tail)
        def _():
            pltpu.sync_copy(e_hbm.at[0, pl.ds(base, PROP_L1)],
                            src_v.at[pl.ds(0, PROP_L1)])
            pltpu.sync_copy(e_hbm.at[1, pl.ds(base, PROP_L1)],
                            dst_v.at[pl.ds(0, PROP_L1)])

        plsc.subcore_barrier()

        for b in range(NBUF):
            pltpu.async_copy(h_hbm.at[src_v.at[b]], rows[b], gsems[b])

        def body(j, carry):
            for b in range(NBUF):
                pltpu.make_async_copy(h_hbm.at[src_v.at[j + b]],
                                      rows[b], gsems[b]).wait()
                pltpu.sync_copy(rows[b], acc.at[dst_v.at[j + b]], add=True)
                pltpu.async_copy(h_hbm.at[src_v.at[j + b + NBUF]],
                                 rows[b], gsems[b])
            return carry

        lax.fori_loop(0, nrows // NBUF - 1, lambda i, c: body(i * NBUF, c), 0)
        j0 = nrows - NBUF
        for b in range(NBUF):
            pltpu.make_async_copy(h_hbm.at[src_v.at[j0 + b]],
                                  rows[b], gsems[b]).wait()
            pltpu.sync_copy(rows[b], acc.at[dst_v.at[j0 + b]], add=True)

        plsc.subcore_barrier()
        pltpu.sync_copy(acc.at[pl.ds(r0, ROWS_PER_SUB)],
                        out_hbm.at[cid, pl.ds(r0, ROWS_PER_SUB)])

    return prop


_prop16 = _make_prop_kernel(HIDDEN)


def _dinv_from_deg(deg_ref):
    deg = deg_ref[0, :, 0:1] + deg_ref[1, :, 0:1]
    return lax.rsqrt(deg + 1.0)


def _tc1_body(x_ref, w_ref, deg_ref, out_ref):
    dinv = _dinv_from_deg(deg_ref)
    h = jnp.dot(x_ref[...], w_ref[...], preferred_element_type=jnp.float32)
    out_ref[0:N_NODES] = h * dinv[0:N_NODES]


_tc1 = pl.pallas_call(
    _tc1_body, out_shape=jax.ShapeDtypeStruct((N_PAD, HIDDEN), jnp.float32))


def _tc2_body(s_ref, h_ref, deg_ref, b_ref, out_ref):
    dinv = lax.rsqrt(deg_ref[0] + deg_ref[1] + 1.0)
    z = dinv * (s_ref[0] + s_ref[1] + h_ref[...]) + b_ref[...]
    out_ref[...] = jnp.maximum(z, 0.0) * dinv


_tc2 = pl.pallas_call(
    _tc2_body, out_shape=jax.ShapeDtypeStruct((N_PAD // 8, 128), jnp.float32))


def _tc3_body(s_ref, h_ref, deg_ref, w_ref, b_ref, out_ref):
    dinv = _dinv_from_deg(deg_ref)
    pz = dinv * (s_ref[0] + s_ref[1] + h_ref[...])
    logits = jnp.dot(pz, w_ref[...], preferred_element_type=jnp.float32) + b_ref[...]
    m = jnp.max(logits, axis=1, keepdims=True)
    lse = jnp.log(jnp.sum(jnp.exp(logits - m), axis=1, keepdims=True)) + m
    out_ref[...] = logits - lse


_tc3 = pl.pallas_call(
    _tc3_body, out_shape=jax.ShapeDtypeStruct((N_PAD, N_CLASSES), jnp.float32))


def kernel(x, edge_index, W1, b1, W2, b2):
    e3 = edge_index.reshape(2, E_ROWS, CHUNK)
    zeros16 = jnp.zeros((N_PAD, HIDDEN), jnp.float32)
    ones = jnp.ones((CHUNK, LANES), jnp.float32)

    degp = _deg_kernel(e3, ones, zeros16)
    h1s = _tc1(x, W1, degp)
    s1p = _prop16(h1s, e3, zeros16)
    z2s = _tc2(s1p.reshape(NC, N_PAD // 8, 128),
               h1s.reshape(N_PAD // 8, 128),
               degp.reshape(NC, N_PAD // 8, 128),
               jnp.tile(b1, 8).reshape(1, 128)).reshape(N_PAD, HIDDEN)
    s2p = _prop16(z2s, e3, zeros16)
    out = _tc3(s2p, z2s, degp, W2, b2.reshape(1, N_CLASSES))
    return out[:N_NODES]
